# baseline (speedup 1.0000x reference)
import functools
import math

import jax
import jax.numpy as jnp
import numpy as np
from jax import lax
from jax.experimental import pallas as pl
from jax.experimental.pallas import tpu as pltpu
from jax.experimental.pallas import tpu_sc as plsc

N, C, H, W = 2, 256, 128, 128
HW = H * W
DIM = 256
PE_DIM = 256
HEADS = 8
HEAD_DIM = DIM // HEADS
WS = 2
NWP = WS * WS
TEMP = 10000.0
SCALE = HEAD_DIM ** (-0.5)

HB = 8
BP = HB * W
NB = HW // BP

NWORK = 32
PER_W = NWP * N * HW // NWORK
CH = 128
NIT = PER_W // CH

PDEG = 10
PROWS = 16

_HI = -65536


def _pos_bias_padded():
    npf = PE_DIM // 2
    eps = 1e-6
    scale = 2 * math.pi
    emb = np.arange(WS, dtype=np.float64) / (WS - 1 + eps) * scale
    dim_t = TEMP ** (2 * (np.arange(npf) // 2) / npf)
    yy, xx = np.meshgrid(emb, emb, indexing="ij")

    def interleave(a):
        out = np.empty_like(a)
        out[..., 0::2] = np.sin(a[..., 0::2])
        out[..., 1::2] = np.cos(a[..., 1::2])
        return out

    posy = interleave(yy[..., None] / dim_t)
    posx = interleave(xx[..., None] / dim_t)
    pos = np.concatenate([posy, posx], axis=-1).reshape(NWP, PE_DIM)
    return np.concatenate([pos, np.zeros((8 - NWP, PE_DIM))], 0).astype(np.float32)


def _pe_poly():
    npf = PE_DIM // 2
    cc = 2 * math.pi / (WS + 1e-6)
    inv_t = TEMP ** (-2 * (np.arange(npf) // 2) / npf)
    u = np.linspace(0.0, 1.0, 4001)
    P = np.zeros((PROWS, npf))
    for d in range(npf):
        arg = u * cc * inv_t[d]
        f = np.sin(arg) if d % 2 == 0 else np.cos(arg)
        P[:PDEG + 1, d] = np.polyfit(u, f, PDEG)[::-1]
    return P.astype(np.float32)


_PB8 = _pos_bias_padded()
_PPE = _pe_poly()


def _pack_kv(kf, vf):
    kb = lax.bitcast_convert_type(kf, jnp.uint32)
    vb = lax.bitcast_convert_type(vf, jnp.uint32)
    kr = (kb + 0x8000) & np.uint32(0xFFFF0000)
    vr = (vb + 0x8000) >> 16
    return lax.bitcast_convert_type(kr | vr, jnp.int32)


def _proj_body(fs_ref, fl_ref, wk_ref, wv_ref, kv_ref, idx_ref):
    b_i = pl.program_id(0)
    wk = wk_ref[...].astype(jnp.bfloat16)
    wv = wv_ref[...].astype(jnp.bfloat16)
    for r in range(HB):
        fsr = fs_ref[0, :, r, :].astype(jnp.bfloat16)
        kf = lax.dot_general(fsr, wk, (((0,), (1,)), ((), ())),
                             preferred_element_type=jnp.float32)
        vf = lax.dot_general(fsr, wv, (((0,), (1,)), ((), ())),
                             preferred_element_type=jnp.float32)
        kv_ref[r] = _pack_kv(kf, vf)

    flt = lax.transpose(fl_ref[0, 0], (1, 0))
    p = b_i * BP + lax.broadcasted_iota(jnp.int32, (1, BP), 1)
    yi = p // W
    xi = p - yi * W
    gx = xi.astype(jnp.float32) + flt[0:1, :]
    gy = yi.astype(jnp.float32) + flt[1:2, :]
    x0 = jnp.floor(gx).astype(jnp.int32)
    y0 = jnp.floor(gy).astype(jnp.int32)
    rows = []
    for dy in (0, 1):
        hy = jnp.clip(y0 + dy, 0, H - 1)
        for dx in (0, 1):
            wx = jnp.clip(x0 + dx, 0, W - 1)
            rows.append(wx + W * hy)
    idx_ref[...] = jnp.concatenate(rows + rows, axis=0)


def _proj_call(feat_supp, fl, Wk, Wv, n_img):
    return pl.pallas_call(
        _proj_body,
        grid=(NB,),
        in_specs=[
            pl.BlockSpec((1, C, HB, W), lambda b: (n_img, 0, b, 0)),
            pl.BlockSpec((1, 1, BP, 2), lambda b: (n_img, b, 0, 0)),
            pl.BlockSpec((DIM, DIM), lambda b: (0, 0)),
            pl.BlockSpec((DIM, DIM), lambda b: (0, 0)),
        ],
        out_specs=[
            pl.BlockSpec((HB, W, DIM), lambda b: (b, 0, 0)),
            pl.BlockSpec((8, BP), lambda b: (0, b)),
        ],
        out_shape=[
            jax.ShapeDtypeStruct((H, W, DIM), jnp.int32),
            jax.ShapeDtypeStruct((8, HW), jnp.int32),
        ],
    )(feat_supp, fl, Wk, Wv)


def _sc_gather(table, idx8, half):
    hoff = half * (HW // 2)
    mesh = plsc.VectorSubcoreMesh(core_axis_name="c", subcore_axis_name="s",
                                  num_cores=2, num_subcores=16)
    per_w = NWP * (HW // 2) // NWORK
    nit = per_w // CH

    @functools.partial(
        pl.kernel,
        mesh=mesh,
        out_type=jax.ShapeDtypeStruct((NWP, HW // 2, DIM), jnp.int32),
        scratch_types=[
            pltpu.VMEM((CH,), jnp.int32),
            pltpu.VMEM((CH,), jnp.int32),
            pltpu.VMEM((CH, DIM), jnp.int32),
            pltpu.VMEM((CH, DIM), jnp.int32),
            pltpu.SemaphoreType.DMA,
            pltpu.SemaphoreType.DMA,
        ],
    )
    def gather_kernel(table_hbm, idx_hbm, out_hbm, idx_a, idx_b, rows_a,
                      rows_b, sem_a, sem_b):
        wid = lax.axis_index("s") * 2 + lax.axis_index("c")
        j = wid // 8
        p0 = (wid - j * 8) * per_w

        pltpu.sync_copy(idx_hbm.at[j, pl.ds(hoff + p0, CH)], idx_a)
        pltpu.async_copy(table_hbm.at[idx_a], rows_a, sem_a)

        def body(i, carry):
            c0 = p0 + 2 * i * CH
            c1 = c0 + CH
            pltpu.sync_copy(idx_hbm.at[j, pl.ds(hoff + c1, CH)], idx_b)
            gb = pltpu.async_copy(table_hbm.at[idx_b], rows_b, sem_b)
            pltpu.make_async_copy(table_hbm.at[idx_a], rows_a, sem_a).wait()
            pltpu.sync_copy(rows_a, out_hbm.at[j, pl.ds(c0, CH)])

            @pl.when(i + 1 < nit // 2)
            def _():
                pltpu.sync_copy(idx_hbm.at[j, pl.ds(hoff + c1 + CH, CH)], idx_a)
                pltpu.async_copy(table_hbm.at[idx_a], rows_a, sem_a)

            gb.wait()
            pltpu.sync_copy(rows_b, out_hbm.at[j, pl.ds(c1, CH)])
            return carry

        lax.fori_loop(0, nit // 2, body, 0)

    return gather_kernel(table, idx8)


def _attn_body_impl(g_ref, fc_ref, fl_ref, pb_ref, ppe_ref, wq_ref, wk_ref,
                    wv_ref, b3_ref, out_ref):
    flt = lax.transpose(fl_ref[0, 0], (1, 0))
    ux = flt[0:1, :]
    uy = flt[1:2, :]
    ux = ux - jnp.floor(ux)
    uy = uy - jnp.floor(uy)

    def powers(u):
        ps = [jnp.ones((1, BP), jnp.float32)]
        for _ in range(PDEG):
            ps.append(ps[-1] * u)
        ps.append(jnp.zeros((PROWS - PDEG - 1, BP), jnp.float32))
        return jnp.concatenate(ps, axis=0)

    vt = jnp.concatenate([powers(uy), powers(ux)], axis=0)

    wq = wq_ref[...]
    cy = lax.dot_general(ppe_ref[...], wq[:, :PE_DIM // 2],
                         (((1,), (1,)), ((), ())),
                         preferred_element_type=jnp.float32)
    cx = lax.dot_general(ppe_ref[...], wq[:, PE_DIM // 2:],
                         (((1,), (1,)), ((), ())),
                         preferred_element_type=jnp.float32)
    c2 = jnp.concatenate([cy, cx], axis=0)

    wqb = wq.astype(jnp.bfloat16)
    qs = []
    for r in range(HB):
        fcr = fc_ref[0, :, r, :].astype(jnp.bfloat16)
        qs.append(lax.dot_general(fcr, wqb, (((0,), (1,)), ((), ())),
                                  preferred_element_type=jnp.float32))
    q = jnp.concatenate(qs, axis=0)
    q = q + lax.dot_general(vt, c2, (((0,), (0,)), ((), ())),
                            preferred_element_type=jnp.float32)
    q = (q + b3_ref[0:1, :]) * SCALE

    kpe = lax.dot_general(pb_ref[...], wk_ref[...], (((1,), (1,)), ((), ())),
                          preferred_element_type=jnp.float32) + b3_ref[1:2, :]
    vpe = lax.dot_general(pb_ref[...], wv_ref[...], (((1,), (1,)), ((), ())),
                          preferred_element_type=jnp.float32) + b3_ref[2:3, :]

    hsel = (lax.broadcasted_iota(jnp.int32, (DIM, HEADS), 0) // HEAD_DIM ==
            lax.broadcasted_iota(jnp.int32, (DIM, HEADS), 1))
    m = hsel.astype(jnp.float32)
    mb = hsel.astype(jnp.bfloat16)

    gs = [g_ref[j] for j in range(NWP)]
    logits = []
    for j in range(NWP):
        kj = lax.bitcast_convert_type(gs[j], jnp.float32) + kpe[j:j + 1, :]
        logits.append(lax.dot_general((q * kj).astype(jnp.bfloat16), mb,
                                      (((1,), (0,)), ((), ())),
                                      preferred_element_type=jnp.float32))
    mx = jnp.maximum(jnp.maximum(logits[0], logits[1]),
                     jnp.maximum(logits[2], logits[3]))
    es = [jnp.exp(l - mx) for l in logits]
    inv = 1.0 / (es[0] + es[1] + es[2] + es[3])
    acc = jnp.zeros((BP, DIM), jnp.float32)
    for j in range(NWP):
        wj = es[j] * inv
        wb = lax.dot_general(wj, m, (((1,), (1,)), ((), ())),
                             preferred_element_type=jnp.float32)
        vj = lax.bitcast_convert_type(gs[j] << 16, jnp.float32) + vpe[j:j + 1, :]
        acc = acc + wb * vj
    acct = lax.transpose(acc, (1, 0))
    for r in range(HB):
        out_ref[0, :, r, :] = acct[:, r * W:(r + 1) * W]


def _attn_call(g3, fc, fl, pb8, ppe, Wq, Wk, Wv, b3, n_img, half, buf=None):
    boff = half * (NB // 2)
    in_specs = [
        pl.BlockSpec((NWP, BP, DIM), lambda b: (0, b, 0)),
        pl.BlockSpec((1, C, HB, W), lambda b: (n_img, 0, b + boff, 0)),
        pl.BlockSpec((1, 1, BP, 2), lambda b: (n_img, b + boff, 0, 0)),
        pl.BlockSpec((8, PE_DIM), lambda b: (0, 0)),
        pl.BlockSpec((PROWS, PE_DIM // 2), lambda b: (0, 0)),
        pl.BlockSpec((DIM, DIM), lambda b: (0, 0)),
        pl.BlockSpec((DIM, DIM), lambda b: (0, 0)),
        pl.BlockSpec((DIM, DIM), lambda b: (0, 0)),
        pl.BlockSpec((8, DIM), lambda b: (0, 0)),
    ]
    args = (g3, fc, fl, pb8, ppe, Wq, Wk, Wv, b3)
    if buf is None:
        body = _attn_body_impl
        aliases = {}
    else:
        def body(buf_ref, *refs):
            _attn_body_impl(*refs)
        in_specs = [pl.BlockSpec(memory_space=pl.ANY)] + in_specs
        args = (buf,) + args
        aliases = {0: 0}
    return pl.pallas_call(
        body,
        grid=(NB // 2,),
        in_specs=in_specs,
        out_specs=pl.BlockSpec((1, DIM, HB, W),
                               lambda b: (n_img, 0, b + boff, 0)),
        out_shape=jax.ShapeDtypeStruct((N, DIM, H, W), jnp.float32),
        input_output_aliases=aliases,
    )(*args)


def kernel(feat_supp, feat_curr, flow, Wq, bq, Wk, bk, Wv, bv):
    fl = flow.reshape(N, NB, BP, 2)
    b3 = jnp.concatenate(
        [bq[None], bk[None], bv[None], jnp.zeros((5, DIM), jnp.float32)], 0)
    pb8 = jnp.asarray(_PB8)
    ppe = jnp.asarray(_PPE)

    kv0, idx0 = _proj_call(feat_supp, fl, Wk, Wv, 0)
    t0 = kv0.reshape(HW, DIM)
    g0a = _sc_gather(t0, idx0, 0)
    kv1, idx1 = _proj_call(feat_supp, fl, Wk, Wv, 1)
    g0b = _sc_gather(t0, idx0, 1)
    t1 = kv1.reshape(HW, DIM)
    g1a = _sc_gather(t1, idx1, 0)
    g1b = _sc_gather(t1, idx1, 1)

    buf = _attn_call(g0a, feat_curr, fl, pb8, ppe, Wq, Wk, Wv, b3, 0, 0)
    buf = _attn_call(g0b, feat_curr, fl, pb8, ppe, Wq, Wk, Wv, b3, 0, 1, buf)
    buf = _attn_call(g1a, feat_curr, fl, pb8, ppe, Wq, Wk, Wv, b3, 1, 0, buf)
    return _attn_call(g1b, feat_curr, fl, pb8, ppe, Wq, Wk, Wv, b3, 1, 1, buf)


# speedup vs baseline: 1.0026x; 1.0026x over previous
import functools
import math

import jax
import jax.numpy as jnp
import numpy as np
from jax import lax
from jax.experimental import pallas as pl
from jax.experimental.pallas import tpu as pltpu
from jax.experimental.pallas import tpu_sc as plsc

N, C, H, W = 2, 256, 128, 128
HW = H * W
DIM = 256
PE_DIM = 256
HEADS = 8
HEAD_DIM = DIM // HEADS
WS = 2
NWP = WS * WS
TEMP = 10000.0
SCALE = HEAD_DIM ** (-0.5)

HB = 8
BP = HB * W
NB = HW // BP

NWORK = 32
PER_W = NWP * N * HW // NWORK
CH = 128
NIT = PER_W // CH

PDEG = 10
PROWS = 16

_HI = -65536


def _pos_bias_padded():
    npf = PE_DIM // 2
    eps = 1e-6
    scale = 2 * math.pi
    emb = np.arange(WS, dtype=np.float64) / (WS - 1 + eps) * scale
    dim_t = TEMP ** (2 * (np.arange(npf) // 2) / npf)
    yy, xx = np.meshgrid(emb, emb, indexing="ij")

    def interleave(a):
        out = np.empty_like(a)
        out[..., 0::2] = np.sin(a[..., 0::2])
        out[..., 1::2] = np.cos(a[..., 1::2])
        return out

    posy = interleave(yy[..., None] / dim_t)
    posx = interleave(xx[..., None] / dim_t)
    pos = np.concatenate([posy, posx], axis=-1).reshape(NWP, PE_DIM)
    return np.concatenate([pos, np.zeros((8 - NWP, PE_DIM))], 0).astype(np.float32)


def _pe_poly():
    npf = PE_DIM // 2
    cc = 2 * math.pi / (WS + 1e-6)
    inv_t = TEMP ** (-2 * (np.arange(npf) // 2) / npf)
    u = np.linspace(0.0, 1.0, 4001)
    P = np.zeros((PROWS, npf))
    for d in range(npf):
        arg = u * cc * inv_t[d]
        f = np.sin(arg) if d % 2 == 0 else np.cos(arg)
        P[:PDEG + 1, d] = np.polyfit(u, f, PDEG)[::-1]
    return P.astype(np.float32)


_PB8 = _pos_bias_padded()
_PPE = _pe_poly()


def _pack_kv(kf, vf):
    kb = lax.bitcast_convert_type(kf, jnp.uint32)
    vb = lax.bitcast_convert_type(vf, jnp.uint32)
    kr = (kb + 0x8000) & np.uint32(0xFFFF0000)
    vr = (vb + 0x8000) >> 16
    return lax.bitcast_convert_type(kr | vr, jnp.int32)


def _proj_body(fs_ref, fl_ref, wk_ref, wv_ref, kv_ref, idx_ref):
    b_i = pl.program_id(0)
    wk = wk_ref[...].astype(jnp.bfloat16)
    wv = wv_ref[...].astype(jnp.bfloat16)
    for r in range(HB):
        fsr = fs_ref[0, :, r, :].astype(jnp.bfloat16)
        kf = lax.dot_general(fsr, wk, (((0,), (1,)), ((), ())),
                             preferred_element_type=jnp.float32)
        vf = lax.dot_general(fsr, wv, (((0,), (1,)), ((), ())),
                             preferred_element_type=jnp.float32)
        kv_ref[r] = _pack_kv(kf, vf)

    flt = lax.transpose(fl_ref[0, 0], (1, 0))
    p = b_i * BP + lax.broadcasted_iota(jnp.int32, (1, BP), 1)
    yi = p // W
    xi = p - yi * W
    gx = xi.astype(jnp.float32) + flt[0:1, :]
    gy = yi.astype(jnp.float32) + flt[1:2, :]
    x0 = jnp.floor(gx).astype(jnp.int32)
    y0 = jnp.floor(gy).astype(jnp.int32)
    rows = []
    for dy in (0, 1):
        hy = jnp.clip(y0 + dy, 0, H - 1)
        for dx in (0, 1):
            wx = jnp.clip(x0 + dx, 0, W - 1)
            rows.append(wx + W * hy)
    idx_ref[...] = jnp.concatenate(rows + rows, axis=0)


def _proj_call(feat_supp, fl, Wk, Wv, n_img):
    return pl.pallas_call(
        _proj_body,
        grid=(NB,),
        in_specs=[
            pl.BlockSpec((1, C, HB, W), lambda b: (n_img, 0, b, 0)),
            pl.BlockSpec((1, 1, BP, 2), lambda b: (n_img, b, 0, 0)),
            pl.BlockSpec((DIM, DIM), lambda b: (0, 0)),
            pl.BlockSpec((DIM, DIM), lambda b: (0, 0)),
        ],
        out_specs=[
            pl.BlockSpec((HB, W, DIM), lambda b: (b, 0, 0)),
            pl.BlockSpec((8, BP), lambda b: (0, b)),
        ],
        out_shape=[
            jax.ShapeDtypeStruct((H, W, DIM), jnp.int32),
            jax.ShapeDtypeStruct((8, HW), jnp.int32),
        ],
    )(feat_supp, fl, Wk, Wv)


def _sc_gather(table, idx8):
    mesh = plsc.VectorSubcoreMesh(core_axis_name="c", subcore_axis_name="s",
                                  num_cores=2, num_subcores=16)
    per_w = NWP * HW // NWORK
    nit = per_w // CH

    @functools.partial(
        pl.kernel,
        mesh=mesh,
        out_type=jax.ShapeDtypeStruct((NWP, HW, DIM), jnp.int32),
        scratch_types=[
            pltpu.VMEM((CH,), jnp.int32),
            pltpu.VMEM((CH,), jnp.int32),
            pltpu.VMEM((CH, DIM), jnp.int32),
            pltpu.VMEM((CH, DIM), jnp.int32),
            pltpu.SemaphoreType.DMA,
            pltpu.SemaphoreType.DMA,
        ],
    )
    def gather_kernel(table_hbm, idx_hbm, out_hbm, idx_a, idx_b, rows_a,
                      rows_b, sem_a, sem_b):
        wid = lax.axis_index("s") * 2 + lax.axis_index("c")
        j = wid // 8
        p0 = (wid - j * 8) * per_w

        pltpu.sync_copy(idx_hbm.at[j, pl.ds(p0, CH)], idx_a)
        pltpu.async_copy(table_hbm.at[idx_a], rows_a, sem_a)

        def body(i, carry):
            c0 = p0 + 2 * i * CH
            c1 = c0 + CH
            pltpu.sync_copy(idx_hbm.at[j, pl.ds(c1, CH)], idx_b)
            gb = pltpu.async_copy(table_hbm.at[idx_b], rows_b, sem_b)
            pltpu.make_async_copy(table_hbm.at[idx_a], rows_a, sem_a).wait()
            pltpu.sync_copy(rows_a, out_hbm.at[j, pl.ds(c0, CH)])

            @pl.when(i + 1 < nit // 2)
            def _():
                pltpu.sync_copy(idx_hbm.at[j, pl.ds(c1 + CH, CH)], idx_a)
                pltpu.async_copy(table_hbm.at[idx_a], rows_a, sem_a)

            gb.wait()
            pltpu.sync_copy(rows_b, out_hbm.at[j, pl.ds(c1, CH)])
            return carry

        lax.fori_loop(0, nit // 2, body, 0)

    return gather_kernel(table, idx8)


def _attn_body_impl(g_ref, fc_ref, fl_ref, pb_ref, ppe_ref, wq_ref, wk_ref,
                    wv_ref, b3_ref, out_ref):
    flt = lax.transpose(fl_ref[0, 0], (1, 0))
    ux = flt[0:1, :]
    uy = flt[1:2, :]
    ux = ux - jnp.floor(ux)
    uy = uy - jnp.floor(uy)

    def powers(u):
        ps = [jnp.ones((1, BP), jnp.float32)]
        for _ in range(PDEG):
            ps.append(ps[-1] * u)
        ps.append(jnp.zeros((PROWS - PDEG - 1, BP), jnp.float32))
        return jnp.concatenate(ps, axis=0)

    vt = jnp.concatenate([powers(uy), powers(ux)], axis=0)

    wq = wq_ref[...]
    cy = lax.dot_general(ppe_ref[...], wq[:, :PE_DIM // 2],
                         (((1,), (1,)), ((), ())),
                         preferred_element_type=jnp.float32)
    cx = lax.dot_general(ppe_ref[...], wq[:, PE_DIM // 2:],
                         (((1,), (1,)), ((), ())),
                         preferred_element_type=jnp.float32)
    c2 = jnp.concatenate([cy, cx], axis=0)

    wqb = wq.astype(jnp.bfloat16)
    qs = []
    for r in range(HB):
        fcr = fc_ref[0, :, r, :].astype(jnp.bfloat16)
        qs.append(lax.dot_general(fcr, wqb, (((0,), (1,)), ((), ())),
                                  preferred_element_type=jnp.float32))
    q = jnp.concatenate(qs, axis=0)
    q = q + lax.dot_general(vt, c2, (((0,), (0,)), ((), ())),
                            preferred_element_type=jnp.float32)
    q = (q + b3_ref[0:1, :]) * SCALE

    kpe = lax.dot_general(pb_ref[...], wk_ref[...], (((1,), (1,)), ((), ())),
                          preferred_element_type=jnp.float32) + b3_ref[1:2, :]
    vpe = lax.dot_general(pb_ref[...], wv_ref[...], (((1,), (1,)), ((), ())),
                          preferred_element_type=jnp.float32) + b3_ref[2:3, :]

    hsel = (lax.broadcasted_iota(jnp.int32, (DIM, HEADS), 0) // HEAD_DIM ==
            lax.broadcasted_iota(jnp.int32, (DIM, HEADS), 1))
    m = hsel.astype(jnp.float32)
    mb = hsel.astype(jnp.bfloat16)

    gs = [g_ref[j] for j in range(NWP)]
    logits = []
    for j in range(NWP):
        kj = lax.bitcast_convert_type(gs[j], jnp.float32) + kpe[j:j + 1, :]
        logits.append(lax.dot_general((q * kj).astype(jnp.bfloat16), mb,
                                      (((1,), (0,)), ((), ())),
                                      preferred_element_type=jnp.float32))
    mx = jnp.maximum(jnp.maximum(logits[0], logits[1]),
                     jnp.maximum(logits[2], logits[3]))
    es = [jnp.exp(l - mx) for l in logits]
    inv = 1.0 / (es[0] + es[1] + es[2] + es[3])
    acc = jnp.zeros((BP, DIM), jnp.float32)
    for j in range(NWP):
        wj = es[j] * inv
        wb = lax.dot_general(wj, m, (((1,), (1,)), ((), ())),
                             preferred_element_type=jnp.float32)
        vj = lax.bitcast_convert_type(gs[j] << 16, jnp.float32) + vpe[j:j + 1, :]
        acc = acc + wb * vj
    acct = lax.transpose(acc, (1, 0))
    for r in range(HB):
        out_ref[0, :, r, :] = acct[:, r * W:(r + 1) * W]


def _attn_call(g3, fc, fl, pb8, ppe, Wq, Wk, Wv, b3, n_img, buf=None):
    in_specs = [
        pl.BlockSpec((NWP, BP, DIM), lambda b: (0, b, 0)),
        pl.BlockSpec((1, C, HB, W), lambda b: (n_img, 0, b, 0)),
        pl.BlockSpec((1, 1, BP, 2), lambda b: (n_img, b, 0, 0)),
        pl.BlockSpec((8, PE_DIM), lambda b: (0, 0)),
        pl.BlockSpec((PROWS, PE_DIM // 2), lambda b: (0, 0)),
        pl.BlockSpec((DIM, DIM), lambda b: (0, 0)),
        pl.BlockSpec((DIM, DIM), lambda b: (0, 0)),
        pl.BlockSpec((DIM, DIM), lambda b: (0, 0)),
        pl.BlockSpec((8, DIM), lambda b: (0, 0)),
    ]
    args = (g3, fc, fl, pb8, ppe, Wq, Wk, Wv, b3)
    if buf is None:
        body = _attn_body_impl
        aliases = {}
    else:
        def body(buf_ref, *refs):
            _attn_body_impl(*refs)
        in_specs = [pl.BlockSpec(memory_space=pl.ANY)] + in_specs
        args = (buf,) + args
        aliases = {0: 0}
    return pl.pallas_call(
        body,
        grid=(NB,),
        in_specs=in_specs,
        out_specs=pl.BlockSpec((1, DIM, HB, W), lambda b: (n_img, 0, b, 0)),
        out_shape=jax.ShapeDtypeStruct((N, DIM, H, W), jnp.float32),
        input_output_aliases=aliases,
    )(*args)


def kernel(feat_supp, feat_curr, flow, Wq, bq, Wk, bk, Wv, bv):
    fl = flow.reshape(N, NB, BP, 2)
    b3 = jnp.concatenate(
        [bq[None], bk[None], bv[None], jnp.zeros((5, DIM), jnp.float32)], 0)
    pb8 = jnp.asarray(_PB8)
    ppe = jnp.asarray(_PPE)

    kv0, idx0 = _proj_call(feat_supp, fl, Wk, Wv, 0)
    g0 = _sc_gather(kv0.reshape(HW, DIM), idx0)
    kv1, idx1 = _proj_call(feat_supp, fl, Wk, Wv, 1)
    g1 = _sc_gather(kv1.reshape(HW, DIM), idx1)

    buf = _attn_call(g0, feat_curr, fl, pb8, ppe, Wq, Wk, Wv, b3, 0)
    return _attn_call(g1, feat_curr, fl, pb8, ppe, Wq, Wk, Wv, b3, 1, buf=buf)
